# Initial kernel scaffold; baseline (speedup 1.0000x reference)
import jax
import jax.numpy as jnp
from jax.experimental import pallas as pl


def kernel(reps):
    raise NotImplementedError("write your pallas kernel here")


# speedup vs baseline: 11.3642x; 11.3642x over previous
import jax
import jax.numpy as jnp
from jax.experimental import pallas as pl

_K = 128
_INT_MIN = -2147483648
_INT_MAX = 2147483647


def _mask_kernel(reps_ref, mask_ref, bin_ref):
    x = reps_ref[...]
    rows, n = x.shape
    b = jax.lax.bitcast_convert_type(x, jnp.int32)
    key = jnp.where(b >= 0, b, (~b) ^ jnp.int32(_INT_MIN))

    def value_step(_, state):
        lo, hi = state
        x_and = lo & hi
        x_xor = lo ^ hi
        mid = x_and + (x_xor >> 1) + (x_xor & 1)
        cnt = jnp.sum((key >= mid).astype(jnp.int32), axis=1, keepdims=True)
        ge = cnt >= _K
        return jnp.where(ge, mid, lo), jnp.where(ge, hi, mid - 1)

    lo0 = jnp.full((rows, 1), _INT_MIN, jnp.int32)
    hi0 = jnp.full((rows, 1), _INT_MAX, jnp.int32)
    thr, _ = jax.lax.fori_loop(0, 32, value_step, (lo0, hi0))

    gt = key > thr
    eq = key == thr
    c_gt = jnp.sum(gt.astype(jnp.int32), axis=1, keepdims=True)
    need = _K - c_gt
    iota = jax.lax.broadcasted_iota(jnp.int32, (rows, n), 1)
    eq_idx = jnp.where(eq, iota, jnp.int32(n))

    def index_step(_, state):
        lo, hi = state
        mid = (lo & hi) + ((lo ^ hi) >> 1)
        cnt = jnp.sum((eq_idx <= mid).astype(jnp.int32), axis=1, keepdims=True)
        ge = cnt >= need
        return jnp.where(ge, lo, mid + 1), jnp.where(ge, mid, hi)

    lo0i = jnp.zeros((rows, 1), jnp.int32)
    hi0i = jnp.full((rows, 1), n - 1, jnp.int32)
    cut, _ = jax.lax.fori_loop(0, 13, index_step, (lo0i, hi0i))

    on = gt | (eq & (iota <= cut))
    mask_ref[...] = jnp.where(on, jnp.float32(1.0), jnp.float32(-1.0))
    bin_ref[...] = on.astype(jnp.bfloat16)


def _overlap_kernel(bi_ref, bj_ref, out_ref):
    out_ref[...] = jax.lax.dot_general(
        bi_ref[...], bj_ref[...], (((1,), (1,)), ((), ())),
        preferred_element_type=jnp.float32)


def kernel(reps):
    m, n = reps.shape
    rows = 256
    mask, binary = pl.pallas_call(
        _mask_kernel,
        grid=(m // rows,),
        in_specs=[pl.BlockSpec((rows, n), lambda i: (i, 0))],
        out_specs=[
            pl.BlockSpec((rows, n), lambda i: (i, 0)),
            pl.BlockSpec((rows, n), lambda i: (i, 0)),
        ],
        out_shape=[
            jax.ShapeDtypeStruct((m, n), jnp.float32),
            jax.ShapeDtypeStruct((m, n), jnp.bfloat16),
        ],
    )(reps)

    bm = 512
    overlaps = pl.pallas_call(
        _overlap_kernel,
        grid=(m // bm, m // bm),
        in_specs=[
            pl.BlockSpec((bm, n), lambda i, j: (i, 0)),
            pl.BlockSpec((bm, n), lambda i, j: (j, 0)),
        ],
        out_specs=pl.BlockSpec((bm, bm), lambda i, j: (i, j)),
        out_shape=jax.ShapeDtypeStruct((m, m), jnp.float32),
    )(binary, binary)
    return (mask, overlaps)


# speedup vs baseline: 15.2789x; 1.3445x over previous
import jax
import jax.numpy as jnp
from jax.experimental import pallas as pl

_K = 128
_INT_MIN = -2147483648
_INT_MAX = 2147483647


def _mask_kernel(reps_ref, mask_ref, bin_ref):
    x = reps_ref[...]
    rows, n = x.shape
    b = jax.lax.bitcast_convert_type(x, jnp.int32)
    key = jnp.where(b >= 0, b, (~b) ^ jnp.int32(_INT_MIN))

    def cond_fn(state):
        i, _, _, done, _ = state
        return (i < 32) & (jnp.min(done) < 1)

    def value_step(state):
        i, lo, hi, done, thr = state
        x_and = lo & hi
        x_xor = lo ^ hi
        mid = x_and + (x_xor >> 1) + (x_xor & 1)
        cnt = jnp.sum((key >= mid).astype(jnp.int32), axis=1, keepdims=True)
        ge = cnt >= _K
        hit = (cnt == _K) & (done < 1)
        thr = jnp.where(hit, mid, thr)
        done = jnp.where(hit, jnp.int32(1), done)
        lo = jnp.where(ge, mid, lo)
        hi = jnp.where(ge, hi, mid - 1)
        return i + 1, lo, hi, done, thr

    lo0 = jnp.full((rows, 1), _INT_MIN, jnp.int32)
    hi0 = jnp.full((rows, 1), _INT_MAX, jnp.int32)
    done0 = jnp.zeros((rows, 1), jnp.int32)
    thr0 = jnp.zeros((rows, 1), jnp.int32)
    _, lo, hi, done, thr = jax.lax.while_loop(
        cond_fn, value_step, (jnp.int32(0), lo0, hi0, done0, thr0))
    done = done > 0

    on_clean = (key >= thr).astype(jnp.float32)

    def tie_path(_):
        t_exact = lo
        gt = key > t_exact
        eq = key == t_exact
        c_gt = jnp.sum(gt.astype(jnp.int32), axis=1, keepdims=True)
        need = _K - c_gt
        iota = jax.lax.broadcasted_iota(jnp.int32, (rows, n), 1)
        eq_idx = jnp.where(eq, iota, jnp.int32(n))

        def index_step(_, state):
            lo_i, hi_i = state
            mid = (lo_i & hi_i) + ((lo_i ^ hi_i) >> 1)
            cnt = jnp.sum((eq_idx <= mid).astype(jnp.int32), axis=1,
                          keepdims=True)
            ge = cnt >= need
            return jnp.where(ge, lo_i, mid + 1), jnp.where(ge, mid, hi_i)

        lo0i = jnp.zeros((rows, 1), jnp.int32)
        hi0i = jnp.full((rows, 1), n - 1, jnp.int32)
        cut, _ = jax.lax.fori_loop(0, 13, index_step, (lo0i, hi0i))
        on_tie = (gt | (eq & (iota <= cut))).astype(jnp.float32)
        return jnp.where(done, on_clean, on_tie)

    on = jax.lax.cond(jnp.all(done),
                      lambda _: on_clean,
                      tie_path,
                      None)
    mask_ref[...] = on * jnp.float32(2.0) - jnp.float32(1.0)
    bin_ref[...] = on.astype(jnp.bfloat16)


def _overlap_kernel(bi_ref, bj_ref, out_ref):
    out_ref[...] = jax.lax.dot_general(
        bi_ref[...], bj_ref[...], (((1,), (1,)), ((), ())),
        preferred_element_type=jnp.float32)


def kernel(reps):
    m, n = reps.shape
    rows = 128
    mask, binary = pl.pallas_call(
        _mask_kernel,
        grid=(m // rows,),
        in_specs=[pl.BlockSpec((rows, n), lambda i: (i, 0))],
        out_specs=[
            pl.BlockSpec((rows, n), lambda i: (i, 0)),
            pl.BlockSpec((rows, n), lambda i: (i, 0)),
        ],
        out_shape=[
            jax.ShapeDtypeStruct((m, n), jnp.float32),
            jax.ShapeDtypeStruct((m, n), jnp.bfloat16),
        ],
    )(reps)

    bm = 512
    overlaps = pl.pallas_call(
        _overlap_kernel,
        grid=(m // bm, m // bm),
        in_specs=[
            pl.BlockSpec((bm, n), lambda i, j: (i, 0)),
            pl.BlockSpec((bm, n), lambda i, j: (j, 0)),
        ],
        out_specs=pl.BlockSpec((bm, bm), lambda i, j: (i, j)),
        out_shape=jax.ShapeDtypeStruct((m, m), jnp.float32),
    )(binary, binary)
    return (mask, overlaps)
